# Initial kernel scaffold; baseline (speedup 1.0000x reference)
import jax
import jax.numpy as jnp
from jax.experimental import pallas as pl


def kernel(x, emb_table, fc_w, fc_b):
    raise NotImplementedError("write your pallas kernel here")


# speedup vs baseline: 2.0622x; 2.0622x over previous
import functools

import jax
import jax.numpy as jnp
from jax import lax
from jax.experimental import pallas as pl
from jax.experimental.pallas import tpu as pltpu
from jax.experimental.pallas import tpu_sc as plsc

VOCAB = 1000000
EMBED = 16
L = 200
B = 16384

VP = 1 << 20
RPB = 1024
NTB = (VP // 8) // RPB

NC = 2
NS = 16
NW = NC * NS
CPT = B // NW
CH = 128
NCH = CPT // CH
NCHG = NW * NCH


def _tbuild_body(emb_ref, w_ref, b_ref, out_ref):
    t = jnp.dot(emb_ref[...], w_ref[...], preferred_element_type=jnp.float32)
    t = t + b_ref[0, 0]

    @pl.when(pl.program_id(0) == 0)
    def _():
        r = lax.broadcasted_iota(jnp.int32, (RPB, 8), 0)
        c = lax.broadcasted_iota(jnp.int32, (RPB, 8), 1)
        out_ref[...] = jnp.where((r == 0) & (c == 0), b_ref[0, 0], t)

    @pl.when(pl.program_id(0) != 0)
    def _():
        out_ref[...] = t


def _tbuild(emb2, wmat, b11):
    return pl.pallas_call(
        _tbuild_body,
        grid=(NTB,),
        in_specs=[
            pl.BlockSpec((RPB, 128), lambda i: (i, 0)),
            pl.BlockSpec((128, 8), lambda i: (0, 0)),
            pl.BlockSpec(memory_space=pltpu.SMEM),
        ],
        out_specs=pl.BlockSpec((RPB, 8), lambda i: (i, 0)),
        out_shape=jax.ShapeDtypeStruct((VP // 8, 8), jnp.float32),
    )(emb2, wmat, b11)


def _sc_pool(t, xprep):
    mesh = plsc.VectorSubcoreMesh(core_axis_name="c", subcore_axis_name="s")

    @functools.partial(
        pl.kernel,
        out_type=jax.ShapeDtypeStruct((B,), jnp.float32),
        mesh=mesh,
        scratch_types=[
            pltpu.VMEM((L * CH,), jnp.int32),
            pltpu.VMEM((L * CH,), jnp.float32),
            pltpu.VMEM((CH,), jnp.float32),
            pltpu.SemaphoreType.DMA,
        ],
    )
    def run(t_hbm, xp_hbm, out_hbm, idx_v, val_v, o_v, sem):
        cid = lax.axis_index("c")
        sid = lax.axis_index("s")
        wid = sid * NC + cid

        def chunk(i, carry):
            g = wid * NCH + i
            pltpu.sync_copy(xp_hbm.at[g], idx_v)
            pltpu.async_copy(t_hbm.at[idx_v], val_v, sem).wait()

            def lbody(l, accs):
                return tuple(
                    accs[k] + val_v[pl.ds(l * CH + k * 16, 16)]
                    for k in range(CH // 16)
                )

            accs = lax.fori_loop(
                0, L, lbody,
                tuple(jnp.zeros((16,), jnp.float32) for _ in range(CH // 16)),
            )
            for k in range(CH // 16):
                z = accs[k] * (1.0 / L)
                o_v[pl.ds(k * 16, 16)] = 1.0 / (1.0 + jnp.exp(-z))
            pltpu.sync_copy(o_v, out_hbm.at[pl.ds(g * CH, CH)])
            return carry

        lax.fori_loop(0, NCH, chunk, 0)

    return run(t, xprep)


def kernel(x, emb_table, fc_w, fc_b):
    w = fc_w.reshape(EMBED).astype(jnp.float32)
    emb2 = jnp.pad(emb_table, ((0, VP - VOCAB), (0, 0))).reshape(VP // 8, 128)
    wmat = jnp.where(
        (jnp.arange(128)[:, None] // EMBED) == jnp.arange(8)[None, :],
        jnp.tile(w, 8)[:, None],
        0.0,
    ).astype(jnp.float32)
    b11 = fc_b.reshape(1, 1).astype(jnp.float32)
    t = _tbuild(emb2, wmat, b11).reshape(VP)
    xprep = (
        x.astype(jnp.int32).reshape(L, NCHG, CH).transpose(1, 0, 2)
        .reshape(NCHG, L * CH)
    )
    return _sc_pool(t, xprep)


# speedup vs baseline: 2.4223x; 1.1746x over previous
import functools

import jax
import jax.numpy as jnp
from jax import lax
from jax.experimental import pallas as pl
from jax.experimental.pallas import tpu as pltpu
from jax.experimental.pallas import tpu_sc as plsc

VOCAB = 1000000
EMBED = 16
L = 200
B = 16384

RPB = 8192
NTB = -(-VOCAB // RPB)
TLEN = NTB * RPB

NC = 2
NS = 16
NW = NC * NS
CPT = B // NW
CH = 256
NCH = CPT // CH


def _tbuild_body(emb_ref, w_ref, b_ref, out_ref):
    t = jnp.sum(emb_ref[...] * w_ref[...], axis=1) + b_ref[0, 0]

    @pl.when(pl.program_id(0) == 0)
    def _():
        r = lax.iota(jnp.int32, RPB)
        out_ref[...] = jnp.where(r == 0, b_ref[0, 0], t)

    @pl.when(pl.program_id(0) != 0)
    def _():
        out_ref[...] = t


def _tbuild(emb, w, b11):
    return pl.pallas_call(
        _tbuild_body,
        grid=(NTB,),
        in_specs=[
            pl.BlockSpec((RPB, EMBED), lambda i: (i, 0)),
            pl.BlockSpec((1, EMBED), lambda i: (0, 0)),
            pl.BlockSpec(memory_space=pltpu.SMEM),
        ],
        out_specs=pl.BlockSpec((RPB,), lambda i: (i,)),
        out_shape=jax.ShapeDtypeStruct((TLEN,), jnp.float32),
    )(emb, w, b11)


def _sc_pool(t, x):
    mesh = plsc.VectorSubcoreMesh(core_axis_name="c", subcore_axis_name="s")

    @functools.partial(
        pl.kernel,
        out_type=jax.ShapeDtypeStruct((B,), jnp.float32),
        mesh=mesh,
        scratch_types=[
            pltpu.VMEM((L * CH,), jnp.int32),
            pltpu.VMEM((L * CH,), jnp.float32),
            pltpu.VMEM((CH,), jnp.float32),
            pltpu.SemaphoreType.DMA,
            pltpu.SemaphoreType.DMA,
        ],
    )
    def run(t_hbm, x_hbm, out_hbm, idx_v, val_v, o_v, isem, gsem):
        cid = lax.axis_index("c")
        sid = lax.axis_index("s")
        wid = sid * NC + cid
        base = wid * CPT

        def chunk(i, carry):
            cbase = base + i * CH
            copies = []
            for l in range(L):
                copies.append(
                    pltpu.async_copy(
                        x_hbm.at[l, pl.ds(cbase, CH)],
                        idx_v.at[pl.ds(l * CH, CH)],
                        isem,
                    )
                )
            for c in copies:
                c.wait()
            pltpu.async_copy(t_hbm.at[idx_v], val_v, gsem).wait()

            def lbody(l, accs):
                return tuple(
                    accs[k] + val_v[pl.ds(l * CH + k * 16, 16)]
                    for k in range(CH // 16)
                )

            accs = lax.fori_loop(
                0, L, lbody,
                tuple(jnp.zeros((16,), jnp.float32) for _ in range(CH // 16)),
            )
            for k in range(CH // 16):
                z = accs[k] * (1.0 / L)
                o_v[pl.ds(k * 16, 16)] = 1.0 / (1.0 + jnp.exp(-z))
            pltpu.sync_copy(o_v, out_hbm.at[pl.ds(cbase, CH)])
            return carry

        lax.fori_loop(0, NCH, chunk, 0)

    return run(t, x)


def kernel(x, emb_table, fc_w, fc_b):
    t = _tbuild(
        emb_table.astype(jnp.float32),
        fc_w.astype(jnp.float32),
        fc_b.reshape(1, 1).astype(jnp.float32),
    )
    return _sc_pool(t, x.astype(jnp.int32))


# speedup vs baseline: 2.6723x; 1.1032x over previous
import functools

import jax
import jax.numpy as jnp
from jax import lax
from jax.experimental import pallas as pl
from jax.experimental.pallas import tpu as pltpu
from jax.experimental.pallas import tpu_sc as plsc

VOCAB = 1000000
EMBED = 16
L = 200
B = 16384

RPB = 8192
NTB = -(-VOCAB // RPB)
TLEN = NTB * RPB

NC = 2
NS = 16
NW = NC * NS
CPT = B // NW
CH = 128
NCH = CPT // CH
TSH = TLEN // NS


def _tbuild_body(emb_ref, w_ref, b_ref, out_ref):
    t = jnp.sum(emb_ref[...] * w_ref[...], axis=1) + b_ref[0, 0]

    @pl.when(pl.program_id(0) == 0)
    def _():
        r = lax.iota(jnp.int32, RPB)
        out_ref[...] = jnp.where(r == 0, b_ref[0, 0], t)

    @pl.when(pl.program_id(0) != 0)
    def _():
        out_ref[...] = t


def _tbuild(emb, w, b11):
    return pl.pallas_call(
        _tbuild_body,
        grid=(NTB,),
        in_specs=[
            pl.BlockSpec((RPB, EMBED), lambda i: (i, 0)),
            pl.BlockSpec((1, EMBED), lambda i: (0, 0)),
            pl.BlockSpec(memory_space=pltpu.SMEM),
        ],
        out_specs=pl.BlockSpec((RPB,), lambda i: (i,)),
        out_shape=jax.ShapeDtypeStruct((TLEN,), jnp.float32),
    )(emb, w, b11)


def _sc_pool(t, x):
    mesh = plsc.VectorSubcoreMesh(core_axis_name="c", subcore_axis_name="s")

    @functools.partial(
        pl.kernel,
        out_type=jax.ShapeDtypeStruct((B,), jnp.float32),
        mesh=mesh,
        scratch_types=[
            pltpu.VMEM_SHARED((TLEN,), jnp.float32),
            pltpu.VMEM((L, CH), jnp.int32),
            pltpu.VMEM((L, CH), jnp.float32),
            pltpu.VMEM((CH,), jnp.float32),
            pltpu.SemaphoreType.DMA,
            pltpu.SemaphoreType.DMA,
            pltpu.SemaphoreType.DMA,
        ],
    )
    def run(t_hbm, x_hbm, out_hbm, t_sh, idx_v, val_v, o_v, tsem, isem, gsem):
        cid = lax.axis_index("c")
        sid = lax.axis_index("s")
        wid = sid * NC + cid
        base = wid * CPT

        tcopy = pltpu.async_copy(
            t_hbm.at[pl.ds(sid * TSH, TSH)],
            t_sh.at[pl.ds(sid * TSH, TSH)],
            tsem,
        )

        def stage(i):
            return pltpu.async_copy(
                x_hbm.at[:, pl.ds(base + i * CH, CH)],
                idx_v,
                isem,
            )

        pending = stage(0)
        tcopy.wait()
        plsc.subcore_barrier()

        for i in range(NCH):
            pending.wait()

            def fire(l, c):
                pltpu.async_copy(
                    t_sh.at[idx_v.at[l]], val_v.at[l], gsem
                )
                return c

            lax.fori_loop(0, L, fire, 0)

            def drain(l, c):
                pltpu.make_async_copy(
                    t_hbm.at[pl.ds(0, CH)], val_v.at[l], gsem
                ).wait()
                return c

            lax.fori_loop(0, L, drain, 0)
            pending = stage(i + 1) if i + 1 < NCH else None

            def lbody(l, accs):
                return tuple(
                    accs[k] + val_v[l, pl.ds(k * 16, 16)]
                    for k in range(CH // 16)
                )

            accs = lax.fori_loop(
                0, L, lbody,
                tuple(jnp.zeros((16,), jnp.float32) for _ in range(CH // 16)),
            )
            for k in range(CH // 16):
                z = accs[k] * (1.0 / L)
                o_v[pl.ds(k * 16, 16)] = 1.0 / (1.0 + jnp.exp(-z))
            pltpu.sync_copy(o_v, out_hbm.at[pl.ds(base + i * CH, CH)])

    return run(t, x)


def kernel(x, emb_table, fc_w, fc_b):
    t = _tbuild(
        emb_table.astype(jnp.float32),
        fc_w.astype(jnp.float32),
        fc_b.reshape(1, 1).astype(jnp.float32),
    )
    return _sc_pool(t, x.astype(jnp.int32))
